# Initial kernel scaffold; baseline (speedup 1.0000x reference)
import jax
import jax.numpy as jnp
from jax.experimental import pallas as pl


def kernel(x, edge_index, W1, b1, W2, b2):
    raise NotImplementedError("write your pallas kernel here")


# speedup vs baseline: 7.6659x; 7.6659x over previous
import functools

import jax
import jax.numpy as jnp
from jax import lax
from jax.experimental import pallas as pl
from jax.experimental.pallas import tpu as pltpu
from jax.experimental.pallas import tpu_sc as plsc

N = 10000
NPAD = 10240
E = 320000
NC, NS = 2, 16
NTILES = NC * NS
CHUNKS = 80
K = 128
EPAD = NTILES * CHUNKS * K
RPT = NPAD // NS
D2 = 128


def _deg_kernel_body(col_hbm, oid_hbm, out_hbm, col_v, oid_v, ones_v, zbuf,
                     acc, sem):
    c = lax.axis_index("c")
    s = lax.axis_index("s")
    wid = s * NC + c
    nzc = NPAD // (NS * K)

    def fill_zero(i, _):
        zbuf[i // 8, pl.ds((i % 8) * 16, 16)] = jnp.zeros((16,), jnp.float32)
        return 0

    lax.fori_loop(0, K * 8, fill_zero, 0)

    def fill_ones(i, _):
        ones_v[i // 8, pl.ds((i % 8) * 16, 16)] = jnp.ones((16,), jnp.float32)
        return 0

    lax.fori_loop(0, K * 8, fill_ones, 0)
    pltpu.sync_copy(col_hbm.at[wid], col_v)
    pltpu.sync_copy(oid_hbm.at[s], oid_v)
    for z in range(nzc):
        pltpu.sync_copy(zbuf, acc.at[oid_v.at[z]])
    plsc.subcore_barrier()

    def body(j, _):
        pltpu.sync_copy(ones_v, acc.at[col_v.at[j]], add=True)
        return 0

    lax.fori_loop(0, CHUNKS, body, 0)
    plsc.subcore_barrier()
    for z in range(nzc):
        pltpu.async_copy(acc.at[oid_v.at[z]], zbuf, sem).wait()
        pltpu.sync_copy(zbuf, out_hbm.at[c, pl.ds(s * RPT + z * K, K)])


def _make_deg_kernel():
    mesh = plsc.VectorSubcoreMesh(core_axis_name="c", subcore_axis_name="s")
    return pl.kernel(
        _deg_kernel_body,
        out_type=jax.ShapeDtypeStruct((NC, NPAD, 128), jnp.float32),
        mesh=mesh,
        scratch_types=[
            pltpu.VMEM((CHUNKS, K), jnp.int32),
            pltpu.VMEM((NPAD // (NS * K), K), jnp.int32),
            pltpu.VMEM((K, 128), jnp.float32),
            pltpu.VMEM((K, 128), jnp.float32),
            pltpu.VMEM_SHARED((NPAD, 128), jnp.float32),
            pltpu.SemaphoreType.DMA,
        ],
    )


def _agg_kernel_body(row_hbm, col_hbm, oid_hbm, y_hbm, out_hbm, row_v, col_v,
                     oid_v, buf, acc, sem, *, depth):
    c = lax.axis_index("c")
    s = lax.axis_index("s")
    wid = s * NC + c
    nz = depth // 16
    nzc = NPAD // (NS * K)

    def fill_zero(i, _):
        buf[i // nz, pl.ds((i % nz) * 16, 16)] = jnp.zeros((16,), jnp.float32)
        return 0

    lax.fori_loop(0, K * nz, fill_zero, 0)
    pltpu.sync_copy(row_hbm.at[wid], row_v)
    pltpu.sync_copy(col_hbm.at[wid], col_v)
    pltpu.sync_copy(oid_hbm.at[s], oid_v)
    for z in range(nzc):
        pltpu.sync_copy(buf, acc.at[oid_v.at[z]])
    plsc.subcore_barrier()

    def body(j, _):
        pltpu.async_copy(y_hbm.at[row_v.at[j]], buf, sem).wait()
        pltpu.sync_copy(buf, acc.at[col_v.at[j]], add=True)
        return 0

    lax.fori_loop(0, CHUNKS, body, 0)
    plsc.subcore_barrier()
    for z in range(nzc):
        pltpu.async_copy(acc.at[oid_v.at[z]], buf, sem).wait()
        pltpu.sync_copy(buf, out_hbm.at[c, pl.ds(s * RPT + z * K, K)])


def _make_agg_kernel(depth):
    mesh = plsc.VectorSubcoreMesh(core_axis_name="c", subcore_axis_name="s")
    return pl.kernel(
        functools.partial(_agg_kernel_body, depth=depth),
        out_type=jax.ShapeDtypeStruct((NC, NPAD, depth), jnp.float32),
        mesh=mesh,
        scratch_types=[
            pltpu.VMEM((CHUNKS, K), jnp.int32),
            pltpu.VMEM((CHUNKS, K), jnp.int32),
            pltpu.VMEM((NPAD // (NS * K), K), jnp.int32),
            pltpu.VMEM((K, depth), jnp.float32),
            pltpu.VMEM_SHARED((NPAD, depth), jnp.float32),
            pltpu.SemaphoreType.DMA,
        ],
    )


def _tca_body(degp_ref, x_ref, w1_ref, dinv_ref, y1_ref):
    deg = degp_ref[0, :, 0:1] + degp_ref[1, :, 0:1] + 1.0
    dinv = lax.rsqrt(deg)
    dinv_ref[...] = dinv
    xw = jnp.dot(x_ref[...], w1_ref[...],
                 preferred_element_type=jnp.float32,
                 precision=lax.Precision.HIGHEST)
    y1_ref[...] = xw * dinv


def _make_tca():
    return pl.pallas_call(
        _tca_body,
        grid=(NPAD // 128,),
        in_specs=[
            pl.BlockSpec((NC, 128, 128), lambda i: (0, i, 0)),
            pl.BlockSpec((128, 128), lambda i: (i, 0)),
            pl.BlockSpec((128, 128), lambda i: (0, 0)),
        ],
        out_specs=[
            pl.BlockSpec((128, 1), lambda i: (i, 0)),
            pl.BlockSpec((128, 128), lambda i: (i, 0)),
        ],
        out_shape=[
            jax.ShapeDtypeStruct((NPAD, 1), jnp.float32),
            jax.ShapeDtypeStruct((NPAD, 128), jnp.float32),
        ],
    )


def _tcb_body(aggp_ref, y1_ref, dinv_ref, b1_ref, w2_ref, y2_ref):
    dinv = dinv_ref[...]
    h = (aggp_ref[0] + aggp_ref[1] + y1_ref[...]) * dinv + b1_ref[...]
    h = jnp.maximum(h, 0.0)
    y2_ref[...] = jnp.dot(h, w2_ref[...],
                          preferred_element_type=jnp.float32,
                          precision=lax.Precision.HIGHEST) * dinv


def _make_tcb():
    return pl.pallas_call(
        _tcb_body,
        grid=(NPAD // 128,),
        in_specs=[
            pl.BlockSpec((NC, 128, 128), lambda i: (0, i, 0)),
            pl.BlockSpec((128, 128), lambda i: (i, 0)),
            pl.BlockSpec((128, 1), lambda i: (i, 0)),
            pl.BlockSpec((1, 128), lambda i: (0, 0)),
            pl.BlockSpec((128, D2), lambda i: (0, 0)),
        ],
        out_specs=pl.BlockSpec((128, D2), lambda i: (i, 0)),
        out_shape=jax.ShapeDtypeStruct((NPAD, D2), jnp.float32),
    )


def _tcc_body(aggp_ref, y2_ref, dinv_ref, b2_ref, out_ref):
    logits = (aggp_ref[0] + aggp_ref[1] + y2_ref[...]) * dinv_ref[...] + b2_ref[...]
    mask = lax.broadcasted_iota(jnp.int32, (128, D2), 1) < 47
    logits = jnp.where(mask, logits, -1e30)
    m = jnp.max(logits, axis=1, keepdims=True)
    e = jnp.where(mask, jnp.exp(logits - m), 0.0)
    out_ref[...] = logits - m - jnp.log(jnp.sum(e, axis=1, keepdims=True))


def _make_tcc():
    return pl.pallas_call(
        _tcc_body,
        grid=(NPAD // 128,),
        in_specs=[
            pl.BlockSpec((NC, 128, D2), lambda i: (0, i, 0)),
            pl.BlockSpec((128, D2), lambda i: (i, 0)),
            pl.BlockSpec((128, 1), lambda i: (i, 0)),
            pl.BlockSpec((1, D2), lambda i: (0, 0)),
        ],
        out_specs=pl.BlockSpec((128, D2), lambda i: (i, 0)),
        out_shape=jax.ShapeDtypeStruct((NPAD, D2), jnp.float32),
    )


def kernel(x, edge_index, W1, b1, W2, b2):
    row = edge_index[0].astype(jnp.int32)
    col = edge_index[1].astype(jnp.int32)
    rowp = jnp.concatenate(
        [row, jnp.zeros((EPAD - E,), jnp.int32)]).reshape(NTILES, CHUNKS, K)
    colp = jnp.concatenate(
        [col, jnp.full((EPAD - E,), NPAD - 1, jnp.int32)]).reshape(NTILES, CHUNKS, K)
    xpad = jnp.pad(x, ((0, NPAD - N), (0, 0)))
    w2p = jnp.pad(W2, ((0, 0), (0, D2 - 47)))
    b1r = b1.reshape(1, 128)
    b2p = jnp.pad(b2, (0, D2 - 47)).reshape(1, D2)

    oidp = jnp.arange(NPAD, dtype=jnp.int32).reshape(NS, NPAD // (NS * K), K)
    degp = _make_deg_kernel()(colp, oidp)
    dinv, y1 = _make_tca()(degp, xpad, W1)
    agg1 = _make_agg_kernel(128)(rowp, colp, oidp, y1)
    y2 = _make_tcb()(agg1, y1, dinv, b1r, w2p)
    agg2 = _make_agg_kernel(D2)(rowp, colp, oidp, y2)
    full = _make_tcc()(agg2, y2, dinv, b2p)
    return full[:N, :47]
